# baseline (speedup 1.0000x reference)
import functools

import jax
import jax.numpy as jnp
from jax import lax
from jax.experimental import pallas as pl
from jax.experimental.pallas import tpu as pltpu
from jax.experimental.pallas import tpu_sc as plsc

_B = 16384
_D = 128
_Q_SIZE = 65536

_info = plsc.get_sparse_core_info()
_NC = _info.num_cores
_NS = _info.num_subcores
_NW = _NC * _NS
_ROWS = _B // _NW

_CHUNK = 64
_NCHUNK = _ROWS // _CHUNK

_mesh = plsc.VectorSubcoreMesh(core_axis_name="c", subcore_axis_name="s")


@functools.partial(
    pl.kernel,
    mesh=_mesh,
    out_type=jax.ShapeDtypeStruct((_B, _D), jnp.float32),
    scratch_types=(
        [pltpu.VMEM((_NCHUNK, _CHUNK, _D), jnp.float32)]
        + [pltpu.SemaphoreType.DMA] * (2 * _NCHUNK)
    ),
)
def _fifo_scatter(feats_hbm, out_hbm, buf, *sems):
    wid = lax.axis_index("s") * _NC + lax.axis_index("c")
    base = wid * _ROWS
    in_sems = sems[:_NCHUNK]
    out_sems = sems[_NCHUNK:]

    def in_copy(i):
        return pltpu.make_async_copy(
            feats_hbm.at[pl.ds(base + i * _CHUNK, _CHUNK)],
            buf.at[i], in_sems[i])

    def out_copy(i):
        return pltpu.make_async_copy(
            buf.at[i],
            out_hbm.at[pl.ds(base + i * _CHUNK, _CHUNK)], out_sems[i])

    for i in range(_NCHUNK):
        in_copy(i).start()
    for i in range(_NCHUNK):
        in_copy(i).wait()
        out_copy(i).start()
    for i in range(_NCHUNK):
        out_copy(i).wait()


def kernel(feats, queue):
    del queue
    return _fifo_scatter(feats)


# speedup vs baseline: 1.0300x; 1.0300x over previous
import functools

import jax
import jax.numpy as jnp
from jax import lax
from jax.experimental import pallas as pl
from jax.experimental.pallas import tpu as pltpu
from jax.experimental.pallas import tpu_sc as plsc

_B = 16384
_D = 128
_Q_SIZE = 65536

_info = plsc.get_sparse_core_info()
_NC = _info.num_cores
_NS = _info.num_subcores
_NW = _NC * _NS
_ROWS = _B // _NW

_CHUNK = 256
_NCHUNK = _ROWS // _CHUNK

_mesh = plsc.VectorSubcoreMesh(core_axis_name="c", subcore_axis_name="s")


@functools.partial(
    pl.kernel,
    mesh=_mesh,
    out_type=jax.ShapeDtypeStruct((_B, _D), jnp.float32),
    scratch_types=(
        [pltpu.VMEM((_NCHUNK, _CHUNK, _D), jnp.float32)]
        + [pltpu.SemaphoreType.DMA] * (2 * _NCHUNK)
    ),
)
def _fifo_scatter(feats_hbm, out_hbm, buf, *sems):
    wid = lax.axis_index("s") * _NC + lax.axis_index("c")
    base = wid * _ROWS
    in_sems = sems[:_NCHUNK]
    out_sems = sems[_NCHUNK:]

    def in_copy(i):
        return pltpu.make_async_copy(
            feats_hbm.at[pl.ds(base + i * _CHUNK, _CHUNK)],
            buf.at[i], in_sems[i])

    def out_copy(i):
        return pltpu.make_async_copy(
            buf.at[i],
            out_hbm.at[pl.ds(base + i * _CHUNK, _CHUNK)], out_sems[i])

    for i in range(_NCHUNK):
        in_copy(i).start()
    for i in range(_NCHUNK):
        in_copy(i).wait()
        out_copy(i).start()
    for i in range(_NCHUNK):
        out_copy(i).wait()


def kernel(feats, queue):
    del queue
    return _fifo_scatter(feats)
